# baseline (speedup 1.0000x reference)
import functools

import jax
import jax.numpy as jnp
from jax.experimental import pallas as pl
from jax.experimental.pallas import tpu as pltpu

_C = 2048


def _sig(x):
    return 0.5 * jnp.tanh(0.5 * x) + 0.5


def _body(feat_ref, featdup_ref, w_iou_ref, b_iou_ref, u_iou_ref, w_f_ref,
          b_f_ref, u_f_ref, w_cls_ref, b_cls_ref, y_ref, hsum_ref, fc_ref,
          *, L, H):
    w_iou = w_iou_ref[...]
    b_iou = b_iou_ref[...]
    u_iou = u_iou_ref[...]
    w_f = w_f_ref[...]
    b_f = b_f_ref[...]
    u_f = u_f_ref[...]
    w_cls = w_cls_ref[...]
    b_cls = b_cls_ref[...]
    for l in range(L - 1, -1, -1):
        M = 1 << l
        cs = min(M, _C)
        for i in range(M // cs):
            r0 = M + i * cs
            x = feat_ref[r0:r0 + cs, :]
            iou = jnp.dot(x, w_iou, preferred_element_type=jnp.float32)
            if l < L - 1:
                hs = hsum_ref[i * cs:(i + 1) * cs, :]
                iou = iou + jnp.dot(hs, u_iou,
                                    preferred_element_type=jnp.float32)
            iou = iou + b_iou
            i_g = _sig(iou[:, :H])
            o_g = _sig(iou[:, H:2 * H])
            u_g = jnp.tanh(iou[:, 2 * H:])
            c_l = i_g * u_g
            if l < L - 1:
                c_l = c_l + fc_ref[i * cs:(i + 1) * cs, :].astype(jnp.float32)
            h_l = o_g * jnp.tanh(c_l)
            h16 = h_l.astype(jnp.bfloat16)
            y = jax.lax.dot_general(
                w_cls, h16, (((1,), (1,)), ((), ())),
                preferred_element_type=jnp.float32)
            y_ref[0:1, r0:r0 + cs] = _sig(y + b_cls)
            if l > 0:
                hp = cs // 2
                xd = featdup_ref[r0:r0 + cs, :]
                xf = jnp.dot(xd, w_f, preferred_element_type=jnp.float32)
                f = _sig(
                    xf + b_f + jnp.dot(h16, u_f,
                                       preferred_element_type=jnp.float32))
                fc2 = (f * c_l).astype(jnp.bfloat16)
                fc_ref[i * hp:(i + 1) * hp, :] = (
                    fc2.reshape(hp, 2, H).sum(axis=1))
                hsum_ref[i * hp:(i + 1) * hp, :] = (
                    h16.reshape(hp, 2, H).sum(axis=1))


def kernel(features, node_evaluation_order, edge_evaluation_order,
           edge_offsets, W_iou, b_iou, U_iou, W_f, b_f, U_f, W_cls, b_cls):
    N, F = features.shape
    H = U_f.shape[0]
    L = (N + 1).bit_length() - 1

    bf16 = jnp.bfloat16
    featp = jnp.concatenate(
        [jnp.zeros((1, F), bf16), features.astype(bf16)], axis=0)
    dup_parts = [jnp.zeros((2, F), bf16)]
    for l in range(1, L):
        dup_parts.append(
            jnp.repeat(featp[1 << (l - 1):1 << l], 2, axis=0))
    featdup = jnp.concatenate(dup_parts, axis=0)
    weights = (W_iou.astype(bf16), b_iou.reshape(1, -1).astype(jnp.float32),
               U_iou.astype(bf16), W_f.astype(bf16),
               b_f.reshape(1, -1).astype(jnp.float32), U_f.astype(bf16),
               W_cls.reshape(1, -1).astype(bf16),
               b_cls.reshape(1, 1).astype(jnp.float32))

    half = max(8, (N + 1) // 4)
    body = functools.partial(_body, L=L, H=H)
    y = pl.pallas_call(
        body,
        out_shape=jax.ShapeDtypeStruct((1, N + 1), jnp.float32),
        scratch_shapes=[
            pltpu.VMEM((half, H), jnp.bfloat16),
            pltpu.VMEM((half, H), jnp.bfloat16),
        ],
    )(featp, featdup, *weights)
    return y.reshape(N + 1, 1)[1:]


# speedup vs baseline: 1.3706x; 1.3706x over previous
import functools

import jax
import jax.numpy as jnp
from jax.experimental import pallas as pl
from jax.experimental.pallas import tpu as pltpu

_C = 2048


def _sig(x):
    return 0.5 * jnp.tanh(0.5 * x) + 0.5


def _body(feat_ref, featdup_ref, w_iou_ref, b_iou_ref, u_iou_ref, w_f_ref,
          b_f_ref, u_f_ref, w_cls_ref, b_cls_ref, y_ref, hsum_ref, fc_ref,
          *, L, H):
    w_iou = w_iou_ref[...]
    b_iou = b_iou_ref[...]
    u_iou = u_iou_ref[...]
    w_f = w_f_ref[...]
    b_f = b_f_ref[...]
    u_f = u_f_ref[...]
    w_cls = w_cls_ref[...]
    b_cls = b_cls_ref[...]
    for l in range(L - 1, -1, -1):
        M = 1 << l
        cs = min(M, _C)
        for i in range(M // cs):
            r0 = M + i * cs
            x = feat_ref[r0:r0 + cs, :]
            iou = jnp.dot(x, w_iou, preferred_element_type=jnp.float32)
            if l < L - 1:
                hs = hsum_ref[i * cs:(i + 1) * cs, :]
                iou = iou + jnp.dot(hs, u_iou,
                                    preferred_element_type=jnp.float32)
            iou = iou + b_iou
            i_g = _sig(iou[:, :H])
            o_g = _sig(iou[:, H:2 * H])
            u_g = jnp.tanh(iou[:, 2 * H:])
            c_l = i_g * u_g
            if l < L - 1:
                c_l = c_l + fc_ref[i * cs:(i + 1) * cs, :].astype(jnp.float32)
            h_l = o_g * jnp.tanh(c_l)
            h16 = h_l.astype(jnp.bfloat16)
            y = jax.lax.dot_general(
                w_cls, h16, (((1,), (1,)), ((), ())),
                preferred_element_type=jnp.float32)
            y_ref[0:1, r0:r0 + cs] = _sig(y + b_cls)
            if l > 0:
                hp = cs // 2
                xd = featdup_ref[r0:r0 + cs, :]
                xf = jnp.dot(xd, w_f, preferred_element_type=jnp.float32)
                f = _sig(
                    xf + b_f + jnp.dot(h16, u_f,
                                       preferred_element_type=jnp.float32))
                fc2 = (f * c_l).astype(jnp.bfloat16)
                fc_ref[i * hp:(i + 1) * hp, :] = (
                    fc2.reshape(hp, 2, H).sum(axis=1))
                hsum_ref[i * hp:(i + 1) * hp, :] = (
                    h16.reshape(hp, 2, H).sum(axis=1))


def kernel(features, node_evaluation_order, edge_evaluation_order,
           edge_offsets, W_iou, b_iou, U_iou, W_f, b_f, U_f, W_cls, b_cls):
    N, F = features.shape
    H = U_f.shape[0]
    L = (N + 1).bit_length() - 1

    bf16 = jnp.bfloat16
    featp = jnp.concatenate(
        [jnp.zeros((1, F), bf16), features.astype(bf16)], axis=0)
    featdup = jnp.repeat(featp[:(N + 1) // 2], 2, axis=0)
    weights = (W_iou.astype(bf16), b_iou.reshape(1, -1).astype(jnp.float32),
               U_iou.astype(bf16), W_f.astype(bf16),
               b_f.reshape(1, -1).astype(jnp.float32), U_f.astype(bf16),
               W_cls.reshape(1, -1).astype(bf16),
               b_cls.reshape(1, 1).astype(jnp.float32))

    half = max(8, (N + 1) // 4)
    body = functools.partial(_body, L=L, H=H)
    y = pl.pallas_call(
        body,
        out_shape=jax.ShapeDtypeStruct((1, N + 1), jnp.float32),
        scratch_shapes=[
            pltpu.VMEM((half, H), jnp.bfloat16),
            pltpu.VMEM((half, H), jnp.bfloat16),
        ],
    )(featp, featdup, *weights)
    return y.reshape(N + 1, 1)[1:]


# speedup vs baseline: 1.5627x; 1.1402x over previous
import functools

import jax
import jax.numpy as jnp
from jax.experimental import pallas as pl
from jax.experimental.pallas import tpu as pltpu

_C = 2048


def _sig(x):
    return 0.5 * jnp.tanh(0.5 * x) + 0.5


def _body(feat_ref, w_iou_ref, b_iou_ref, u_iou_ref, w_f_ref, b_f_ref,
          u_f_ref, w_cls_ref, b_cls_ref, y_ref, hsum_ref, fc_ref, *, L, H):
    w_iou = w_iou_ref[...]
    b_iou = b_iou_ref[...]
    u_iou = u_iou_ref[...]
    w_f = w_f_ref[...]
    b_f = b_f_ref[...]
    u_f = u_f_ref[...]
    w_cls = w_cls_ref[...]
    b_cls = b_cls_ref[...]
    for l in range(L - 1, -1, -1):
        M = 1 << l
        cs = min(M, _C)
        for i in range(M // cs):
            r0 = M + i * cs
            x = feat_ref[r0:r0 + cs, :]
            iou = jnp.dot(x, w_iou, preferred_element_type=jnp.float32)
            if l < L - 1:
                hs = hsum_ref[i * cs:(i + 1) * cs, :]
                iou = iou + jnp.dot(hs, u_iou,
                                    preferred_element_type=jnp.float32)
            iou = iou + b_iou
            i_g = _sig(iou[:, :H])
            o_g = _sig(iou[:, H:2 * H])
            u_g = jnp.tanh(iou[:, 2 * H:])
            c_l = i_g * u_g
            if l < L - 1:
                c_l = c_l + fc_ref[i * cs:(i + 1) * cs, :].astype(jnp.float32)
            h_l = o_g * jnp.tanh(c_l)
            h16 = h_l.astype(jnp.bfloat16)
            y = jax.lax.dot_general(
                w_cls, h16, (((1,), (1,)), ((), ())),
                preferred_element_type=jnp.float32)
            y_ref[0:1, r0:r0 + cs] = _sig(y + b_cls)
            if l > 0:
                hp = cs // 2
                p0 = M // 2 + i * hp
                xp = feat_ref[p0:p0 + hp, :]
                xf = jnp.dot(xp, w_f, preferred_element_type=jnp.float32)
                xrep = jnp.broadcast_to(
                    (xf + b_f)[:, None, :], (hp, 2, H)).reshape(cs, H)
                f = _sig(
                    xrep + jnp.dot(h16, u_f,
                                   preferred_element_type=jnp.float32))
                fc2 = (f * c_l).astype(jnp.bfloat16)
                fc_ref[i * hp:(i + 1) * hp, :] = (
                    fc2.reshape(hp, 2, H).sum(axis=1))
                hsum_ref[i * hp:(i + 1) * hp, :] = (
                    h16.reshape(hp, 2, H).sum(axis=1))


def kernel(features, node_evaluation_order, edge_evaluation_order,
           edge_offsets, W_iou, b_iou, U_iou, W_f, b_f, U_f, W_cls, b_cls):
    N, F = features.shape
    H = U_f.shape[0]
    L = (N + 1).bit_length() - 1

    bf16 = jnp.bfloat16
    featp = jnp.concatenate(
        [jnp.zeros((1, F), bf16), features.astype(bf16)], axis=0)

    weights = (W_iou.astype(bf16), b_iou.reshape(1, -1).astype(jnp.float32),
               U_iou.astype(bf16), W_f.astype(bf16),
               b_f.reshape(1, -1).astype(jnp.float32), U_f.astype(bf16),
               W_cls.reshape(1, -1).astype(bf16),
               b_cls.reshape(1, 1).astype(jnp.float32))

    half = max(8, (N + 1) // 4)
    body = functools.partial(_body, L=L, H=H)
    y = pl.pallas_call(
        body,
        out_shape=jax.ShapeDtypeStruct((1, N + 1), jnp.float32),
        scratch_shapes=[
            pltpu.VMEM((half, H), jnp.bfloat16),
            pltpu.VMEM((half, H), jnp.bfloat16),
        ],
    )(featp, *weights)
    return y.reshape(N + 1, 1)[1:]


# speedup vs baseline: 1.7388x; 1.1127x over previous
import functools

import jax
import jax.numpy as jnp
from jax.experimental import pallas as pl
from jax.experimental.pallas import tpu as pltpu

_C = 2048


def _sig(x):
    return 0.5 * jnp.tanh(0.5 * x) + 0.5


def _body(feat_ref, w_iou_ref, b_iou_ref, u_iou_ref, w_f_ref, b_f_ref,
          u_f_ref, w_cls_ref, b_cls_ref, y_ref, hsum_ref, fc_ref, *, L, H):
    w_iou = w_iou_ref[...]
    b_iou = b_iou_ref[...]
    u_iou = u_iou_ref[...]
    w_f = w_f_ref[...]
    b_f = b_f_ref[...]
    u_f = u_f_ref[...]
    w_cls = w_cls_ref[...]
    b_cls = b_cls_ref[...]
    for l in range(L - 1, -1, -1):
        M = 1 << l
        cs = min(M, _C)
        for i in range(M // cs):
            r0 = M + i * cs
            x = feat_ref[r0:r0 + cs, :]
            iou = jnp.dot(x, w_iou, preferred_element_type=jnp.float32)
            if l < L - 1:
                hs = hsum_ref[i * cs:(i + 1) * cs, :]
                iou = iou + jnp.dot(hs, u_iou,
                                    preferred_element_type=jnp.float32)
            iou = iou + b_iou
            i_g = _sig(iou[:, :H])
            o_g = _sig(iou[:, H:2 * H])
            u_g = jnp.tanh(iou[:, 2 * H:])
            c_l = i_g * u_g
            if l < L - 1:
                c_l = c_l + fc_ref[i * cs:(i + 1) * cs, :]
            h_l = o_g * jnp.tanh(c_l)
            h16 = h_l.astype(jnp.bfloat16)
            y = jax.lax.dot_general(
                w_cls, h16, (((1,), (1,)), ((), ())),
                preferred_element_type=jnp.float32)
            y_ref[0:1, r0:r0 + cs] = _sig(y + b_cls)
            if l > 0:
                hp = cs // 2
                p0 = M // 2 + i * hp
                xp = feat_ref[p0:p0 + hp, :]
                xf = jnp.dot(xp, w_f, preferred_element_type=jnp.float32)
                xf = xf + b_f
                xrep = jnp.broadcast_to(
                    xf[:, None, :], (hp, 2, H)).reshape(cs, H)
                f = _sig(
                    xrep + jnp.dot(h16, u_f,
                                   preferred_element_type=jnp.float32))
                fc2 = f * c_l
                fc_ref[i * hp:(i + 1) * hp, :] = (
                    fc2.reshape(hp, 2, H).sum(axis=1))
                hsum_ref[i * hp:(i + 1) * hp, :] = (
                    h_l.reshape(hp, 2, H).sum(axis=1).astype(hsum_ref.dtype))


def kernel(features, node_evaluation_order, edge_evaluation_order,
           edge_offsets, W_iou, b_iou, U_iou, W_f, b_f, U_f, W_cls, b_cls):
    N, F = features.shape
    H = U_f.shape[0]
    L = (N + 1).bit_length() - 1

    bf16 = jnp.bfloat16
    featp = jnp.concatenate(
        [jnp.zeros((1, F), bf16), features.astype(bf16)], axis=0)

    weights = (W_iou.astype(bf16), b_iou.reshape(1, -1).astype(jnp.float32),
               U_iou.astype(bf16), W_f.astype(bf16),
               b_f.reshape(1, -1).astype(jnp.float32), U_f.astype(bf16),
               W_cls.reshape(1, -1).astype(bf16),
               b_cls.reshape(1, 1).astype(jnp.float32))

    half = max(8, (N + 1) // 4)
    body = functools.partial(_body, L=L, H=H)
    y = pl.pallas_call(
        body,
        out_shape=jax.ShapeDtypeStruct((1, N + 1), jnp.float32),
        scratch_shapes=[
            pltpu.VMEM((half, H), jnp.bfloat16),
            pltpu.VMEM((half, H), jnp.float32),
        ],
    )(featp, *weights)
    return y.reshape(N + 1, 1)[1:]
